# baseline (speedup 1.0000x reference)
import functools

import jax
import jax.numpy as jnp
import numpy as np
from jax import lax
from jax.experimental import pallas as pl
from jax.experimental.pallas import tpu as pltpu
from jax.experimental.pallas import tpu_sc as plsc

_GRID = 0.5
_SCALE_MULT = 3.0
_PC_MIN = np.array([-50.0, -50.0, -5.0], dtype=np.float32)

_NCELL_X = 8
_NCELL_Y = 8
_NCELLS = _NCELL_X * _NCELL_Y
_CELL_VOX = 25
_LCAP = 192
_NG = 1024
_NPTS = 8192
_NCLS = 17
_NW = 32
_PPW = _NPTS // _NW
_LIST_LEN = _NCELLS * _LCAP + 4 * _NG + 16
_OVF_BASE = _NCELLS * _LCAP
_TRASH = _NCELLS * _LCAP + 4 * _NG


def _sc_kernel(px_h, py_h, pz_h, tix_h, tiy_h, tiz_h, cell_h,
               gtab_h, sem_h, eg_h, epos_h, misc_h, out_h,
               px_v, py_v, pz_v, tix_v, tiy_v, tiz_v, cell_v,
               gtab_v, sem_v, eg_v, epos_v, misc_v, list_v, out_v):
    wid = lax.axis_index("s") * 2 + lax.axis_index("c")
    base = wid * _PPW

    pltpu.sync_copy(px_h.at[pl.ds(base, _PPW)], px_v)
    pltpu.sync_copy(py_h.at[pl.ds(base, _PPW)], py_v)
    pltpu.sync_copy(pz_h.at[pl.ds(base, _PPW)], pz_v)
    pltpu.sync_copy(tix_h.at[pl.ds(base, _PPW)], tix_v)
    pltpu.sync_copy(tiy_h.at[pl.ds(base, _PPW)], tiy_v)
    pltpu.sync_copy(tiz_h.at[pl.ds(base, _PPW)], tiz_v)
    pltpu.sync_copy(cell_h.at[pl.ds(base, _PPW)], cell_v)
    pltpu.sync_copy(gtab_h, gtab_v)
    pltpu.sync_copy(sem_h, sem_v)
    pltpu.sync_copy(eg_h, eg_v)
    pltpu.sync_copy(epos_h, epos_v)
    pltpu.sync_copy(misc_h, misc_v)

    lane = lax.iota(jnp.int32, 16)

    def build(e, _):
        eg_l = eg_v[pl.ds(e * 16, 16)]
        ep_l = epos_v[pl.ds(e * 16, 16)]
        ok = ep_l >= 0
        plsc.store_scatter(list_v, [jnp.where(ok, ep_l, _TRASH)],
                           eg_l.astype(jnp.float32))
        return 0

    lax.fori_loop(0, (4 * _NG) // 16, build, 0)

    mi = misc_v[...]
    n_ovf = jnp.max(mi)

    def eval_block(k_base, g_l, valid, px, py, pz, tx, ty, tz, acc):
        mx = plsc.load_gather(gtab_v, [g_l])
        my = plsc.load_gather(gtab_v, [g_l + _NG])
        mz = plsc.load_gather(gtab_v, [g_l + 2 * _NG])
        jx = plsc.load_gather(gtab_v, [g_l + 3 * _NG])
        jy = plsc.load_gather(gtab_v, [g_l + 4 * _NG])
        jz = plsc.load_gather(gtab_v, [g_l + 5 * _NG])
        rr = plsc.load_gather(gtab_v, [g_l + 6 * _NG])
        ax = plsc.load_gather(gtab_v, [g_l + 7 * _NG])
        ay = plsc.load_gather(gtab_v, [g_l + 8 * _NG])
        az = plsc.load_gather(gtab_v, [g_l + 9 * _NG])
        c0 = plsc.load_gather(gtab_v, [g_l + 10 * _NG])
        dx = px - mx
        dy = py - my
        dz = pz - mz
        power = ax * (dx * dx) + ay * (dy * dy) + az * (dz * dz) + c0
        inside = ((jnp.abs(tx - jx) <= rr)
                  & (jnp.abs(ty - jy) <= rr)
                  & (jnp.abs(tz - jz) <= rr) & valid)
        w = jnp.where(inside, jnp.exp(power), 0.0)
        sbase = g_l * _NCLS
        new_acc = []
        for c in range(_NCLS):
            s = plsc.load_gather(sem_v, [sbase + c])
            new_acc.append(acc[c] + w * s)
        return tuple(new_acc)

    def group(g, _):
        sl = pl.ds(g * 16, 16)
        px, py, pz = px_v[sl], py_v[sl], pz_v[sl]
        tx, ty, tz = tix_v[sl], tiy_v[sl], tiz_v[sl]
        cells = cell_v[sl]
        counts = plsc.load_gather(gtab_v, [cells + 11 * _NG])
        counts_i = counts.astype(jnp.int32)
        kmax = jnp.max(counts_i)
        lbase = cells * _LCAP

        zero = jnp.zeros((16,), jnp.float32)
        acc0 = tuple(zero for _ in range(_NCLS))

        def main_body(k, acc):
            g_l = plsc.load_gather(list_v, [lbase + k]).astype(jnp.int32)
            valid = k < counts_i
            g_l = jnp.where(valid, g_l, 0)
            return eval_block(k, g_l, valid, px, py, pz, tx, ty, tz, acc)

        acc = lax.fori_loop(0, kmax, main_body, acc0)

        def ovf_body(k, acc):
            g_l = plsc.load_gather(
                list_v, [jnp.full((16,), _OVF_BASE, jnp.int32) + k]
            ).astype(jnp.int32)
            valid = jnp.full((16,), True)
            return eval_block(k, g_l, valid, px, py, pz, tx, ty, tz, acc)

        acc = lax.fori_loop(0, n_ovf, ovf_body, acc)

        rows = g * 16 + lane
        for c in range(_NCLS):
            plsc.store_scatter(out_v, [rows, jnp.full((16,), c, jnp.int32)],
                               acc[c])
        return 0

    lax.fori_loop(0, _PPW // 16, group, 0)

    pltpu.sync_copy(out_v, out_h.at[pl.ds(base, _PPW)])


def kernel(pts, means3D, opacities, semantics, scales, cov3D):
    p = pts[0]
    m = means3D[0].astype(jnp.float32)
    op = opacities[0].astype(jnp.float32)
    sem = semantics[0].astype(jnp.float32)
    sc = scales[0]
    cov = cov3D[0].astype(jnp.float32)
    pc_min = jnp.asarray(_PC_MIN)

    pint = ((p - pc_min) / _GRID).astype(jnp.int32)
    mint = ((m - pc_min) / _GRID).astype(jnp.int32)
    radii = jnp.ceil(jnp.max(sc, axis=-1) * _SCALE_MULT / _GRID).astype(jnp.int32)

    cell_p = (jnp.clip(pint[:, 0] // _CELL_VOX, 0, _NCELL_X - 1) * _NCELL_Y
              + jnp.clip(pint[:, 1] // _CELL_VOX, 0, _NCELL_Y - 1)).astype(jnp.int32)

    x0 = jnp.clip((mint[:, 0] - radii) // _CELL_VOX, 0, _NCELL_X - 1)
    x1 = jnp.clip((mint[:, 0] + radii) // _CELL_VOX, 0, _NCELL_X - 1)
    y0 = jnp.clip((mint[:, 1] - radii) // _CELL_VOX, 0, _NCELL_Y - 1)
    y1 = jnp.clip((mint[:, 1] + radii) // _CELL_VOX, 0, _NCELL_Y - 1)
    s0 = x0 * _NCELL_Y + y0
    s1 = x0 * _NCELL_Y + y1
    s2 = x1 * _NCELL_Y + y0
    s3 = x1 * _NCELL_Y + y1
    s1 = jnp.where(s1 == s0, -1, s1)
    s2 = jnp.where(s2 == s0, -1, s2)
    s3 = jnp.where((s3 == s0) | (s3 == s1) | (s3 == s2), -1, s3)
    ecell = jnp.stack([s0, s1, s2, s3], axis=1).reshape(-1)
    eg = jnp.repeat(jnp.arange(_NG, dtype=jnp.int32), 4)

    onehot = (ecell[:, None] == jnp.arange(_NCELLS, dtype=jnp.int32)[None, :]
              ).astype(jnp.float32)
    tril = jnp.tril(jnp.ones((_NG, _NG), jnp.float32), k=-1)
    og = onehot.reshape(_NG, 4, _NCELLS).sum(axis=1)
    prev_g = jnp.dot(tril, og)
    slot_prefix = jnp.cumsum(onehot.reshape(_NG, 4, _NCELLS), axis=1) \
        - onehot.reshape(_NG, 4, _NCELLS)
    pos = (prev_g[:, None, :] + slot_prefix).reshape(4 * _NG, _NCELLS)
    pos_e = jnp.sum(pos * onehot, axis=1).astype(jnp.int32)
    counts_raw = og.sum(axis=0).astype(jnp.int32)
    counts = jnp.minimum(counts_raw, _LCAP)

    valid_e = ecell >= 0
    is_ovf = valid_e & (pos_e >= _LCAP)
    opos = jnp.cumsum(is_ovf.astype(jnp.int32)) - is_ovf.astype(jnp.int32)
    epos = jnp.where(valid_e,
                     jnp.where(is_ovf, _OVF_BASE + opos,
                               ecell * _LCAP + pos_e),
                     -1).astype(jnp.int32)
    n_ovf = jnp.sum(is_ovf.astype(jnp.int32))
    misc = jnp.zeros((16,), jnp.int32).at[0].set(n_ovf)

    cd = jnp.stack([cov[:, 0, 0], cov[:, 1, 1], cov[:, 2, 2]], axis=0)
    gtab = jnp.concatenate([
        m[:, 0], m[:, 1], m[:, 2],
        mint[:, 0].astype(jnp.float32), mint[:, 1].astype(jnp.float32),
        mint[:, 2].astype(jnp.float32), radii.astype(jnp.float32),
        -0.5 * cd[0], -0.5 * cd[1], -0.5 * cd[2], jnp.log(op),
        jnp.zeros((_NG,), jnp.float32).at[:_NCELLS].set(
            counts.astype(jnp.float32)),
    ]).astype(jnp.float32)

    mesh = plsc.VectorSubcoreMesh(core_axis_name="c", subcore_axis_name="s")
    fn = functools.partial(
        pl.kernel, _sc_kernel, mesh=mesh,
        compiler_params=pltpu.CompilerParams(needs_layout_passes=False),
        out_type=jax.ShapeDtypeStruct((_NPTS, _NCLS), jnp.float32),
        scratch_types=[
            pltpu.VMEM((_PPW,), jnp.float32),
            pltpu.VMEM((_PPW,), jnp.float32),
            pltpu.VMEM((_PPW,), jnp.float32),
            pltpu.VMEM((_PPW,), jnp.float32),
            pltpu.VMEM((_PPW,), jnp.float32),
            pltpu.VMEM((_PPW,), jnp.float32),
            pltpu.VMEM((_PPW,), jnp.int32),
            pltpu.VMEM((12 * _NG,), jnp.float32),
            pltpu.VMEM((_NG * _NCLS,), jnp.float32),
            pltpu.VMEM((4 * _NG,), jnp.int32),
            pltpu.VMEM((4 * _NG,), jnp.int32),
            pltpu.VMEM((16,), jnp.int32),
            pltpu.VMEM((_LIST_LEN,), jnp.float32),
            pltpu.VMEM((_PPW, _NCLS), jnp.float32),
        ],
    )()
    out = fn(p[:, 0], p[:, 1], p[:, 2],
             pint[:, 0].astype(jnp.float32), pint[:, 1].astype(jnp.float32),
             pint[:, 2].astype(jnp.float32), cell_p,
             gtab, sem.reshape(-1), eg, epos, misc)
    return out
